# Initial kernel scaffold; baseline (speedup 1.0000x reference)
import jax
import jax.numpy as jnp
from jax.experimental import pallas as pl


def kernel(in0, in1):
    raise NotImplementedError("write your pallas kernel here")


# speedup vs baseline: 1.6569x; 1.6569x over previous
import functools

import jax
import jax.numpy as jnp
from jax import lax
from jax.experimental import pallas as pl
from jax.experimental.pallas import tpu as pltpu
from jax.experimental.pallas import tpu_sc as plsc

E = 128
S0, S1, SO = 4, 3, 4
B = 50000
NC, NS = 2, 16
NW = NC * NS
C = 32
G = -(-B // C)
T = -(-G // NW)

OUT_PATHS = (
    ((0, 0, 1.0), (3, 1, 0.4)),
    ((1, 0, 0.5), (0, 1, 0.2)),
    ((2, 1, -0.3), (1, 2, 1.1)),
    ((3, 2, 0.7), (2, 0, -0.9)),
)

_mesh = plsc.VectorSubcoreMesh(core_axis_name="c", subcore_axis_name="s")


@functools.partial(
    pl.kernel,
    mesh=_mesh,
    out_type=jax.ShapeDtypeStruct((B, SO * E), jnp.float32),
    scratch_types=[
        pltpu.VMEM((C, S0 * E), jnp.float32),
        pltpu.VMEM((C, S1 * E), jnp.float32),
        pltpu.VMEM((C, SO * E), jnp.float32),
    ],
)
def _sc_poly(in0_hbm, in1_hbm, out_hbm, x0_v, x1_v, o_v):
    wid = lax.axis_index("s") * NC + lax.axis_index("c")

    def chunk_body(t, carry):
        g = jnp.minimum(wid + t * NW, G - 1)
        start = jnp.minimum(g * C, B - C)
        pltpu.sync_copy(in0_hbm.at[pl.ds(start, C)], x0_v)
        pltpu.sync_copy(in1_hbm.at[pl.ds(start, C)], x1_v)

        def row_body(r, rc):
            for v in range(E // 16):
                o = v * 16
                a = [x0_v[r, pl.ds(i * E + o, 16)] for i in range(S0)]
                b = [x1_v[r, pl.ds(j * E + o, 16)] for j in range(S1)]
                for k, ((i1, j1, c1), (i2, j2, c2)) in enumerate(OUT_PATHS):
                    acc = jnp.float32(c1) * (a[i1] * b[j1]) \
                        + jnp.float32(c2) * (a[i2] * b[j2])
                    o_v[r, pl.ds(k * E + o, 16)] = acc
            return rc

        lax.fori_loop(0, C, row_body, 0)
        pltpu.sync_copy(o_v, out_hbm.at[pl.ds(start, C)])
        return carry

    lax.fori_loop(0, T, chunk_body, 0)


def kernel(in0, in1):
    return _sc_poly(in0, in1)


# speedup vs baseline: 3.2623x; 1.9689x over previous
import functools

import jax
import jax.numpy as jnp
from jax import lax
from jax.experimental import pallas as pl
from jax.experimental.pallas import tpu as pltpu
from jax.experimental.pallas import tpu_sc as plsc

E = 128
S0, S1, SO = 4, 3, 4
B = 50000
NC, NS = 2, 16
NW = NC * NS
C = 40
G = B // C
T = 2 * (-(-G // (2 * NW)))

OUT_PATHS = (
    ((0, 0, 1.0), (3, 1, 0.4)),
    ((1, 0, 0.5), (0, 1, 0.2)),
    ((2, 1, -0.3), (1, 2, 1.1)),
    ((3, 2, 0.7), (2, 0, -0.9)),
)

_mesh = plsc.VectorSubcoreMesh(core_axis_name="c", subcore_axis_name="s")


@functools.partial(
    pl.kernel,
    mesh=_mesh,
    out_type=jax.ShapeDtypeStruct((B, SO * E), jnp.float32),
    scratch_types=[
        pltpu.VMEM((2, C, S0 * E), jnp.float32),
        pltpu.VMEM((2, C, S1 * E), jnp.float32),
        pltpu.VMEM((2, C, SO * E), jnp.float32),
        pltpu.SemaphoreType.DMA((2,)),
        pltpu.SemaphoreType.DMA((2,)),
        pltpu.SemaphoreType.DMA((2,)),
    ],
)
def _sc_poly(in0_hbm, in1_hbm, out_hbm, x0_v, x1_v, o_v, s0, s1, so):
    wid = lax.axis_index("s") * NC + lax.axis_index("c")

    def g_of(t):
        return wid + t * NW

    def issue_in(t, b):
        @pl.when(g_of(t) < G)
        def _():
            st = g_of(t) * C
            pltpu.make_async_copy(
                in0_hbm.at[pl.ds(st, C)], x0_v.at[b], s0.at[b]).start()
            pltpu.make_async_copy(
                in1_hbm.at[pl.ds(st, C)], x1_v.at[b], s1.at[b]).start()

    def wait_in(t, b):
        @pl.when(g_of(t) < G)
        def _():
            pltpu.make_async_copy(
                in0_hbm.at[pl.ds(0, C)], x0_v.at[b], s0.at[b]).wait()
            pltpu.make_async_copy(
                in1_hbm.at[pl.ds(0, C)], x1_v.at[b], s1.at[b]).wait()

    def issue_out(t, b):
        @pl.when(g_of(t) < G)
        def _():
            st = g_of(t) * C
            pltpu.make_async_copy(
                o_v.at[b], out_hbm.at[pl.ds(st, C)], so.at[b]).start()

    def wait_out(t, b):
        @pl.when((t >= 0) & (g_of(t) < G))
        def _():
            pltpu.make_async_copy(
                o_v.at[b], out_hbm.at[pl.ds(0, C)], so.at[b]).wait()

    def compute(t, b):
        @pl.when(g_of(t) < G)
        def _():
            def row_body(r, rc):
                for v in range(E // 16):
                    o = v * 16
                    a = [x0_v[b, r, pl.ds(i * E + o, 16)] for i in range(S0)]
                    c = [x1_v[b, r, pl.ds(j * E + o, 16)] for j in range(S1)]
                    for k, ((i1, j1, c1), (i2, j2, c2)) in enumerate(OUT_PATHS):
                        acc = jnp.float32(c1) * (a[i1] * c[j1]) \
                            + jnp.float32(c2) * (a[i2] * c[j2])
                        o_v[b, r, pl.ds(k * E + o, 16)] = acc
                return rc

            lax.fori_loop(0, C, row_body, 0)

    issue_in(0, 0)
    issue_in(1, 1)

    def pipe_body(tt, carry):
        for b in range(2):
            t = 2 * tt + b
            wait_in(t, b)
            wait_out(t - 2, b)
            compute(t, b)
            issue_out(t, b)
            issue_in(t + 2, b)
        return carry

    lax.fori_loop(0, T // 2, pipe_body, 0)
    wait_out(T - 2, 0)
    wait_out(T - 1, 1)


def kernel(in0, in1):
    return _sc_poly(in0, in1)


# speedup vs baseline: 4.2139x; 1.2917x over previous
import functools

import jax
import jax.numpy as jnp
from jax import lax
from jax.experimental import pallas as pl
from jax.experimental.pallas import tpu as pltpu
from jax.experimental.pallas import tpu_sc as plsc

E = 128
S0, S1, SO = 4, 3, 4
B = 50000
NC, NS = 2, 16
NW = NC * NS
C = 40
G = B // C
T = 2 * (-(-G // (2 * NW)))

OUT_PATHS = (
    ((0, 0, 1.0), (3, 1, 0.4)),
    ((1, 0, 0.5), (0, 1, 0.2)),
    ((2, 1, -0.3), (1, 2, 1.1)),
    ((3, 2, 0.7), (2, 0, -0.9)),
)

_mesh = plsc.VectorSubcoreMesh(core_axis_name="c", subcore_axis_name="s")


@functools.partial(
    pl.kernel,
    mesh=_mesh,
    out_type=jax.ShapeDtypeStruct((B, SO * E), jnp.float32),
    scratch_types=[
        pltpu.VMEM((2, C, S0 * E), jnp.float32),
        pltpu.VMEM((2, C, S1 * E), jnp.float32),
        pltpu.VMEM((2, C, SO * E), jnp.float32),
        pltpu.SemaphoreType.DMA((2,)),
        pltpu.SemaphoreType.DMA((2,)),
        pltpu.SemaphoreType.DMA((2,)),
    ],
)
def _sc_poly(in0_hbm, in1_hbm, out_hbm, x0_v, x1_v, o_v, s0, s1, so):
    wid = lax.axis_index("s") * NC + lax.axis_index("c")

    def g_of(t):
        return wid + t * NW

    def issue_in(t, b):
        @pl.when(g_of(t) < G)
        def _():
            st = g_of(t) * C
            pltpu.make_async_copy(
                in0_hbm.at[pl.ds(st, C)], x0_v.at[b], s0.at[b]).start()
            pltpu.make_async_copy(
                in1_hbm.at[pl.ds(st, C)], x1_v.at[b], s1.at[b]).start()

    def wait_in(t, b):
        @pl.when(g_of(t) < G)
        def _():
            pltpu.make_async_copy(
                in0_hbm.at[pl.ds(0, C)], x0_v.at[b], s0.at[b]).wait()
            pltpu.make_async_copy(
                in1_hbm.at[pl.ds(0, C)], x1_v.at[b], s1.at[b]).wait()

    def issue_out(t, b):
        @pl.when(g_of(t) < G)
        def _():
            st = g_of(t) * C
            pltpu.make_async_copy(
                o_v.at[b], out_hbm.at[pl.ds(st, C)], so.at[b]).start()

    def wait_out(t, b):
        @pl.when((t >= 0) & (g_of(t) < G))
        def _():
            pltpu.make_async_copy(
                o_v.at[b], out_hbm.at[pl.ds(0, C)], so.at[b]).wait()

    def compute(t, b):
        @pl.when(g_of(t) < G)
        def _():
            @plsc.parallel_loop(0, C, unroll=2)
            def row_body(r):
                for v in range(E // 16):
                    o = v * 16
                    a = [x0_v[b, r, pl.ds(i * E + o, 16)] for i in range(S0)]
                    c = [x1_v[b, r, pl.ds(j * E + o, 16)] for j in range(S1)]
                    for k, ((i1, j1, c1), (i2, j2, c2)) in enumerate(OUT_PATHS):
                        acc = jnp.float32(c1) * (a[i1] * c[j1]) \
                            + jnp.float32(c2) * (a[i2] * c[j2])
                        o_v[b, r, pl.ds(k * E + o, 16)] = acc

    issue_in(0, 0)
    issue_in(1, 1)

    def pipe_body(tt, carry):
        for b in range(2):
            t = 2 * tt + b
            wait_in(t, b)
            wait_out(t - 2, b)
            compute(t, b)
            issue_out(t, b)
            issue_in(t + 2, b)
        return carry

    lax.fori_loop(0, T // 2, pipe_body, 0)
    wait_out(T - 2, 0)
    wait_out(T - 1, 1)


def kernel(in0, in1):
    return _sc_poly(in0, in1)
